# baseline (speedup 1.0000x reference)
import functools

import jax
import jax.numpy as jnp
from jax import lax
from jax.experimental import pallas as pl
from jax.experimental.pallas import tpu as pltpu
from jax.experimental.pallas import tpu_sc as plsc

N = 16384
ARITY = 2
D_C = 32
D_P = 32
D_ATOM = 64

NC = 2
NS = 16
L = 16
NW = NC * NS
TPW = N // NW
IDX_CHUNK = 128
CT_CH = ARITY * TPW // IDX_CHUNK
PE_CH = TPW // IDX_CHUNK


def _sc_gather(x_domain, tri2d, pid2d, constant_table, predicate_table):
    mesh = plsc.VectorSubcoreMesh(
        core_axis_name="c", subcore_axis_name="s", num_cores=NC, num_subcores=NS
    )

    @functools.partial(
        pl.kernel,
        out_type=(
            jax.ShapeDtypeStruct((ARITY * N, D_C), jnp.float32),
            jax.ShapeDtypeStruct((N, D_P), jnp.float32),
        ),
        mesh=mesh,
        compiler_params=pltpu.CompilerParams(
            needs_layout_passes=False, use_tc_tiling_on_sc=False),
        scratch_types=[
            pltpu.VMEM((N,), jnp.int32),
            pltpu.VMEM((CT_CH, IDX_CHUNK), jnp.int32),
            pltpu.VMEM((CT_CH, IDX_CHUNK), jnp.int32),
            pltpu.VMEM((PE_CH, IDX_CHUNK), jnp.int32),
            pltpu.VMEM((ARITY * TPW, D_C), jnp.float32),
            pltpu.VMEM((TPW, D_P), jnp.float32),
            pltpu.SemaphoreType.DMA,
        ],
    )
    def k(xd_hbm, tri_hbm, pid_hbm, ctab_hbm, ptab_hbm, ct_out, pe_out,
          xdom_v, tri_v, idx2_v, pidx_v, ct_v, pe_v, sem):
        wid = lax.axis_index("s") * NC + lax.axis_index("c")
        pltpu.sync_copy(xd_hbm, xdom_v)
        pltpu.sync_copy(tri_hbm.at[pl.ds(wid * CT_CH, CT_CH)], tri_v)
        pltpu.sync_copy(pid_hbm.at[pl.ds(wid * PE_CH, PE_CH)], pidx_v)

        for j in range(CT_CH):
            for i in range(IDX_CHUNK // L):
                iv = tri_v[j, pl.ds(i * L, L)]
                idx2_v[j, pl.ds(i * L, L)] = plsc.load_gather(xdom_v, [iv])

        cps = []
        for j in range(CT_CH):
            cps.append(pltpu.async_copy(
                ctab_hbm.at[idx2_v.at[j]],
                ct_v.at[pl.ds(j * IDX_CHUNK, IDX_CHUNK)], sem))
        for j in range(PE_CH):
            cps.append(pltpu.async_copy(
                ptab_hbm.at[pidx_v.at[j]],
                pe_v.at[pl.ds(j * IDX_CHUNK, IDX_CHUNK)], sem))
        for c in cps:
            c.wait()

        pltpu.sync_copy(ct_v, ct_out.at[pl.ds(ARITY * TPW * wid, ARITY * TPW)])
        pltpu.sync_copy(pe_v, pe_out.at[pl.ds(TPW * wid, TPW)])

    return k(x_domain, tri2d, pid2d, constant_table, predicate_table)


BN = 2048


def _mlp_body(pe_ref, ct_ref, w1p_ref, w1c_ref, b1_ref, w2_ref, b2_ref,
              emb_ref, out_ref):
    h = jnp.dot(pe_ref[...], w1p_ref[...], preferred_element_type=jnp.float32)
    h = h + jnp.dot(ct_ref[...], w1c_ref[...], preferred_element_type=jnp.float32)
    h = jnp.maximum(h + b1_ref[...], 0.0)
    emb_ref[...] = h
    o = jnp.dot(h, w2_ref[...], preferred_element_type=jnp.float32) + b2_ref[...]
    out_ref[...] = jax.nn.sigmoid(o)


def _tc_mlp(pe, ct, W1, b1, W2, b2):
    w1p = W1[:D_P]
    w1c = W1[D_P:]
    emb, out = pl.pallas_call(
        _mlp_body,
        grid=(N // BN,),
        in_specs=[
            pl.BlockSpec((BN, D_P), lambda i: (i, 0)),
            pl.BlockSpec((BN, ARITY * D_C), lambda i: (i, 0)),
            pl.BlockSpec((D_P, D_ATOM), lambda i: (0, 0)),
            pl.BlockSpec((ARITY * D_C, D_ATOM), lambda i: (0, 0)),
            pl.BlockSpec((1, D_ATOM), lambda i: (0, 0)),
            pl.BlockSpec((D_ATOM, 1), lambda i: (0, 0)),
            pl.BlockSpec((1, 1), lambda i: (0, 0)),
        ],
        out_specs=[
            pl.BlockSpec((BN, D_ATOM), lambda i: (i, 0)),
            pl.BlockSpec((BN, 1), lambda i: (i, 0)),
        ],
        out_shape=[
            jax.ShapeDtypeStruct((N, D_ATOM), jnp.float32),
            jax.ShapeDtypeStruct((N, 1), jnp.float32),
        ],
    )(pe, ct, w1p, w1c, b1.reshape(1, D_ATOM), W2, b2.reshape(1, 1))
    return emb, out


def kernel(X_domain, triplet_idx, pred_ids, constant_table, predicate_table,
           W1, b1, W2, b2):
    tri2d = triplet_idx.astype(jnp.int32).reshape(NW * CT_CH, IDX_CHUNK)
    pid2d = pred_ids.astype(jnp.int32).reshape(NW * PE_CH, IDX_CHUNK)
    ct, pe = _sc_gather(X_domain.astype(jnp.int32), tri2d, pid2d,
                        constant_table, predicate_table)
    ct = ct.reshape(N, ARITY * D_C)
    emb, out = _tc_mlp(pe, ct, W1, b1, W2, b2)
    return out.reshape(N, 1, 1), emb
